# baseline (speedup 1.0000x reference)
import functools

import jax
import jax.numpy as jnp
from jax import lax
from jax.experimental import pallas as pl
from jax.experimental.pallas import tpu as pltpu
from jax.experimental.pallas import tpu_sc as plsc

BATCH = 16384
FIELDS = 4
EMBED = 64
ROWS = FIELDS * EMBED
RT = ROWS // 8
CT = BATCH // 128
NC = 2
NS = 16
NW = NC * NS
CT_PER_W = CT // NW


def _sc_body(y_hbm, f_hbm, b_hbm, out_hbm, ybuf, fbuf, bbuf, obuf):
    wid = lax.axis_index("s") * NC + lax.axis_index("c")
    pltpu.sync_copy(b_hbm, bbuf)
    for t0 in range(CT_PER_W):
        tc = wid * CT_PER_W + t0
        pltpu.sync_copy(y_hbm.at[:, tc], ybuf)
        pltpu.sync_copy(f_hbm.at[tc], fbuf)

        def g_body(g, carry):
            sl = pl.ds(g * 16, 16)
            bias = bbuf[...]
            inter = jnp.zeros((16,), jnp.float32)
            sq = jnp.zeros((16,), jnp.float32)
            for e_hi in range(8):
                for b in range(8):
                    v0 = ybuf[e_hi, b, sl]
                    v1 = ybuf[8 + e_hi, b, sl]
                    v2 = ybuf[16 + e_hi, b, sl]
                    v3 = ybuf[24 + e_hi, b, sl]
                    s = (v0 + v1) + (v2 + v3)
                    inter = inter + s * s
                    sq = sq + ((v0 * v0 + v1 * v1) + (v2 * v2 + v3 * v3))
            ft = (fbuf[0, sl] + fbuf[1, sl]) + (fbuf[2, sl] + fbuf[3, sl])
            obuf[sl] = bias + ft + 0.5 * (inter - sq)
            return carry

        lax.fori_loop(0, 8, g_body, 0)
        pltpu.sync_copy(obuf, out_hbm.at[pl.ds(tc * 128, 128)])


@functools.partial(
    pl.kernel,
    out_type=jax.ShapeDtypeStruct((BATCH,), jnp.float32),
    mesh=plsc.VectorSubcoreMesh(core_axis_name="c", subcore_axis_name="s"),
    scratch_types=[
        pltpu.VMEM((RT, 8, 128), jnp.float32),
        pltpu.VMEM((FIELDS, 128), jnp.float32),
        pltpu.VMEM((16,), jnp.float32),
        pltpu.VMEM((128,), jnp.float32),
    ],
)
def _sc_fm(y_hbm, f_hbm, b_hbm, out_hbm, ybuf, fbuf, bbuf, obuf):
    _sc_body(y_hbm, f_hbm, b_hbm, out_hbm, ybuf, fbuf, bbuf, obuf)


def kernel(first_embeddings, second_embeddings, bias):
    xt = jnp.transpose(second_embeddings, (1, 2, 0)).reshape(ROWS, BATCH)
    y4 = jnp.transpose(xt.reshape(RT, 8, CT, 128), (0, 2, 1, 3))
    ft = jnp.transpose(first_embeddings, (1, 0))
    f3 = jnp.transpose(ft.reshape(FIELDS, CT, 128), (1, 0, 2))
    b16 = jnp.broadcast_to(bias, (16,))
    return _sc_fm(y4, f3, b16)
